# baseline (device time: 51719 ns/iter reference)
import jax
import jax.numpy as jnp
from jax import lax
from jax.experimental import pallas as pl
from jax.experimental.pallas import tpu as pltpu

N_DEV = 4
SQ = 1024
SKV = 1024
D_MODEL = 1024
HQ_PER = 8
DH = 128
HD_PER = HQ_PER * DH
CHUNK = SQ // N_DEV
SCALE = 0.08838834764831843
NEG_INF = -1e9


def kernel(x, Wq, K_ext, V_ext, Wo):
    def body(x_hbm, wq_hbm, k_hbm, v_hbm, wo_hbm, out_ref,
             xf, wqf, wof, kf, vf, xb, wqb, wob, kb, vb,
             qbuf, cbuf, pbuf, sbufA, rbufA, sbufB, rbufB,
             copy_sems, sendA, recvA, sendB, recvB):
        my_pos = lax.axis_index("i")

        dmas = []
        for src, dst, i in (
            (x_hbm.at[0], xf, 0),
            (wq_hbm.at[:, pl.ds(my_pos * HD_PER, HD_PER)], wqf, 1),
            (wo_hbm.at[pl.ds(my_pos * HD_PER, HD_PER), :], wof, 2),
        ):
            c = pltpu.make_async_copy(src, dst, copy_sems.at[i])
            c.start()
            dmas.append(c)
        kv_dmas = []
        for h in range(HQ_PER):
            ck = pltpu.make_async_copy(
                k_hbm.at[0, :, h, :], kf.at[h], copy_sems.at[3]
            )
            cv = pltpu.make_async_copy(
                v_hbm.at[0, :, h, :], vf.at[h], copy_sems.at[4]
            )
            ck.start()
            cv.start()
            kv_dmas.append((ck, cv))

        barrier_sem = pltpu.get_barrier_semaphore()
        for j in range(N_DEV - 1):
            peer = lax.rem(my_pos + 1 + j, N_DEV)
            pl.semaphore_signal(barrier_sem, inc=1, device_id=(peer,),
                                device_id_type=pl.DeviceIdType.MESH)
        pl.semaphore_wait(barrier_sem, N_DEV - 1)

        dmas[0].wait()
        dmas[1].wait()
        xb[:, :] = xf[:, :].astype(jnp.bfloat16)
        wqb[:, :] = wqf[:, :].astype(jnp.bfloat16)
        qbuf[:, :] = jnp.dot(
            xb[:, :], wqb[:, :], preferred_element_type=jnp.float32
        ).astype(jnp.bfloat16)

        for ck, cv in kv_dmas:
            ck.wait()
            cv.wait()
        for h in range(HQ_PER):
            kb[h] = kf[h].astype(jnp.bfloat16)
            vb[h] = vf[h].astype(jnp.bfloat16)
        dmas[2].wait()
        wob[:, :] = wof[:, :].astype(jnp.bfloat16)

        GLOB = 128


        def attn_glob_rows(nrows):
            for h in range(HQ_PER):
                qh = qbuf[0:nrows, h * DH:(h + 1) * DH]
                s = lax.dot_general(
                    qh, kb[h], (((1,), (1,)), ((), ())),
                    preferred_element_type=jnp.float32,
                ) * SCALE
                w = jnp.exp(s)
                w = w * (1.0 / jnp.sum(w, axis=1, keepdims=True))
                ctx_h = jnp.dot(w.astype(jnp.bfloat16), vb[h],
                                preferred_element_type=jnp.float32)
                cbuf[0:nrows, h * DH:(h + 1) * DH] = ctx_h.astype(jnp.bfloat16)

        def attn_win(off, r0, nrows, width, bs, with_glob):
            qi = lax.broadcasted_iota(jnp.int32, (nrows, width), 0) + off + r0
            kib = lax.broadcasted_iota(jnp.int32, (nrows, width), 1) + bs
            mask_b = (jnp.abs(qi - kib) <= 128) | (kib < 32)
            if with_glob:
                kig = lax.broadcasted_iota(jnp.int32, (nrows, GLOB), 1)
                mask_g = (kig < 32) & (bs != 0)
            for h in range(HQ_PER):
                qh = qbuf[pl.ds(off + r0, nrows), h * DH:(h + 1) * DH]
                s_b = lax.dot_general(
                    qh, kb[h, pl.ds(bs, width), :], (((1,), (1,)), ((), ())),
                    preferred_element_type=jnp.float32,
                ) * SCALE
                wb = jnp.exp(jnp.where(mask_b, s_b, NEG_INF))
                denom = jnp.sum(wb, axis=1, keepdims=True)
                if with_glob:
                    s_g = lax.dot_general(
                        qh, kb[h, 0:GLOB, :], (((1,), (1,)), ((), ())),
                        preferred_element_type=jnp.float32,
                    ) * SCALE
                    wg = jnp.exp(jnp.where(mask_g, s_g, NEG_INF))
                    denom = denom + jnp.sum(wg, axis=1, keepdims=True)
                r = 1.0 / denom
                ctx_h = jnp.dot(
                    (wb * r).astype(jnp.bfloat16), vb[h, pl.ds(bs, width), :],
                    preferred_element_type=jnp.float32,
                )
                if with_glob:
                    ctx_h = ctx_h + jnp.dot(
                        (wg * r).astype(jnp.bfloat16), vb[h, 0:GLOB, :],
                        preferred_element_type=jnp.float32,
                    )
                cbuf[r0:r0 + nrows, h * DH:(h + 1) * DH] = (
                    ctx_h.astype(jnp.bfloat16)
                )

        def compute_chunk(off):

            @pl.when(off == 0)
            def _chunk0():
                attn_glob_rows(32)
                attn_win(0, 32, CHUNK - 32, 512, 0, with_glob=False)

            @pl.when(off != 0)
            def _sparse():
                bs = jnp.minimum(off - 128, SKV - 512)
                attn_win(off, 0, CHUNK, 512, bs, with_glob=True)

            return jnp.dot(cbuf[:, :], wob[:, :],
                           preferred_element_type=jnp.float32)

        HALF = CHUNK // 2

        def compute_half(off2):

            @pl.when(off2 == 0)
            def _half0():
                attn_glob_rows(32)
                attn_win(0, 32, HALF - 32, 256, 0, with_glob=False)

            @pl.when(off2 != 0)
            def _sparse():
                bs = jnp.minimum(off2 - 128, SKV - 384)
                attn_win(off2, 0, HALF, 384, bs, with_glob=True)

            return jnp.dot(cbuf[0:HALF, :], wob[:, :],
                           preferred_element_type=jnp.float32)

        sendsA = []
        for j in range(N_DEV - 1):
            target = lax.rem(my_pos + 1 + j, N_DEV)
            sbufA[j] = compute_chunk(target * CHUNK).astype(jnp.bfloat16)
            rdma = pltpu.make_async_remote_copy(
                src_ref=sbufA.at[j],
                dst_ref=rbufA.at[2 - j],
                send_sem=sendA.at[j],
                recv_sem=recvA.at[2 - j],
                device_id=(target,),
                device_id_type=pl.DeviceIdType.MESH,
            )
            rdma.start()
            sendsA.append(rdma)

        def ag_send(half, j, target):
            rdma = pltpu.make_async_remote_copy(
                src_ref=sbufB.at[pl.ds(half * HALF, HALF)],
                dst_ref=rbufB.at[2 - j, pl.ds(half * HALF, HALF)],
                send_sem=sendB.at[2 * j + half],
                recv_sem=recvB.at[2 * (2 - j) + half],
                device_id=(target,),
                device_id_type=pl.DeviceIdType.MESH,
            )
            rdma.start()
            return rdma

        sendsB = []
        off_own = my_pos * CHUNK
        for half in range(2):
            ph = compute_half(off_own + half * HALF)
            if half == 0:
                for r in (2, 1, 0):
                    recv = pltpu.make_async_remote_copy(
                        src_ref=sbufA.at[0],
                        dst_ref=rbufA.at[r],
                        send_sem=sendA.at[0],
                        recv_sem=recvA.at[r],
                        device_id=(my_pos,),
                        device_id_type=pl.DeviceIdType.MESH,
                    )
                    recv.wait_recv()
            for r in range(3):
                ph = ph + rbufA[r, pl.ds(half * HALF, HALF)].astype(jnp.float32)
            sbufB[pl.ds(half * HALF, HALF)] = ph.astype(jnp.bfloat16)
            for j in range(N_DEV - 1):
                target = lax.rem(my_pos + 1 + j, N_DEV)
                sendsB.append(ag_send(half, j, target))
        out_ref[0, pl.ds(off_own, CHUNK), :] = sbufB[:, :]

        for half in range(2):
            for r in (2, 1, 0):
                recv = pltpu.make_async_remote_copy(
                    src_ref=sbufB.at[pl.ds(half * HALF, HALF)],
                    dst_ref=rbufB.at[r, pl.ds(half * HALF, HALF)],
                    send_sem=sendB.at[0],
                    recv_sem=recvB.at[2 * r + half],
                    device_id=(my_pos,),
                    device_id_type=pl.DeviceIdType.MESH,
                )
                recv.wait_recv()
                src_dev = lax.rem(my_pos + 1 + r, N_DEV)
                out_ref[0, pl.ds(src_dev * CHUNK + half * HALF, HALF), :] = (
                    rbufB[r, pl.ds(half * HALF, HALF)]
                )

        for rdma in sendsA + sendsB:
            rdma.wait_send()

    return pl.pallas_call(
        body,
        out_shape=jax.ShapeDtypeStruct((1, SQ, D_MODEL), jnp.bfloat16),
        in_specs=[pl.BlockSpec(memory_space=pl.ANY)] * 5,
        out_specs=pl.BlockSpec(memory_space=pltpu.VMEM),
        scratch_shapes=[
            pltpu.VMEM((SQ, D_MODEL), jnp.float32),
            pltpu.VMEM((D_MODEL, HD_PER), jnp.float32),
            pltpu.VMEM((HD_PER, D_MODEL), jnp.float32),
            pltpu.VMEM((HQ_PER, SKV, DH), jnp.float32),
            pltpu.VMEM((HQ_PER, SKV, DH), jnp.float32),
            pltpu.VMEM((SQ, D_MODEL), jnp.bfloat16),
            pltpu.VMEM((D_MODEL, HD_PER), jnp.bfloat16),
            pltpu.VMEM((HD_PER, D_MODEL), jnp.bfloat16),
            pltpu.VMEM((HQ_PER, SKV, DH), jnp.bfloat16),
            pltpu.VMEM((HQ_PER, SKV, DH), jnp.bfloat16),
            pltpu.VMEM((SQ, HD_PER), jnp.bfloat16),
            pltpu.VMEM((CHUNK, HD_PER), jnp.bfloat16),
            pltpu.VMEM((CHUNK, D_MODEL), jnp.float32),
            pltpu.VMEM((N_DEV - 1, CHUNK, D_MODEL), jnp.bfloat16),
            pltpu.VMEM((N_DEV - 1, CHUNK, D_MODEL), jnp.bfloat16),
            pltpu.VMEM((CHUNK, D_MODEL), jnp.bfloat16),
            pltpu.VMEM((N_DEV - 1, CHUNK, D_MODEL), jnp.bfloat16),
            pltpu.SemaphoreType.DMA((5,)),
            pltpu.SemaphoreType.DMA((N_DEV - 1,)),
            pltpu.SemaphoreType.DMA((N_DEV - 1,)),
            pltpu.SemaphoreType.DMA((2 * (N_DEV - 1),)),
            pltpu.SemaphoreType.DMA((2 * (N_DEV - 1),)),
        ],
        compiler_params=pltpu.CompilerParams(
            collective_id=0, vmem_limit_bytes=64 * 1024 * 1024
        ),
    )(x, Wq, K_ext, V_ext, Wo)


# device time: 51271 ns/iter; 1.0087x vs baseline; 1.0087x over previous
import jax
import jax.numpy as jnp
from jax import lax
from jax.experimental import pallas as pl
from jax.experimental.pallas import tpu as pltpu

N_DEV = 4
SQ = 1024
SKV = 1024
D_MODEL = 1024
HQ_PER = 8
DH = 128
HD_PER = HQ_PER * DH
CHUNK = SQ // N_DEV
SCALE = 0.08838834764831843
NEG_INF = -1e9


def kernel(x, Wq, K_ext, V_ext, Wo):
    def body(x_hbm, wq_hbm, k_hbm, v_hbm, wo_hbm, out_ref,
             xf, wqf, wof, kf, vf, xb, wqb, wob, kb, vb,
             qbuf, cbuf, pbuf, sbufA, rbufA, sbufB, rbufB,
             copy_sems, sendA, recvA, sendB, recvB):
        my_pos = lax.axis_index("i")

        dmas = []
        for src, dst, i in (
            (x_hbm.at[0], xf, 0),
            (wq_hbm.at[:, pl.ds(my_pos * HD_PER, HD_PER)], wqf, 1),
            (wo_hbm.at[pl.ds(my_pos * HD_PER, HD_PER), :], wof, 2),
        ):
            c = pltpu.make_async_copy(src, dst, copy_sems.at[i])
            c.start()
            dmas.append(c)
        kv_dmas = []
        for h in range(HQ_PER):
            ck = pltpu.make_async_copy(
                k_hbm.at[0, :, h, :], kf.at[h], copy_sems.at[3]
            )
            cv = pltpu.make_async_copy(
                v_hbm.at[0, :, h, :], vf.at[h], copy_sems.at[4]
            )
            ck.start()
            cv.start()
            kv_dmas.append((ck, cv))

        barrier_sem = pltpu.get_barrier_semaphore()
        for j in range(N_DEV - 1):
            peer = lax.rem(my_pos + 1 + j, N_DEV)
            pl.semaphore_signal(barrier_sem, inc=1, device_id=(peer,),
                                device_id_type=pl.DeviceIdType.MESH)
        pl.semaphore_wait(barrier_sem, N_DEV - 1)

        dmas[0].wait()
        dmas[1].wait()
        xb[:, :] = xf[:, :].astype(jnp.bfloat16)
        wqb[:, :] = wqf[:, :].astype(jnp.bfloat16)
        qbuf[:, :] = jnp.dot(
            xb[:, :], wqb[:, :], preferred_element_type=jnp.float32
        ).astype(jnp.bfloat16)

        for ck, cv in kv_dmas:
            ck.wait()
            cv.wait()
        for h in range(HQ_PER):
            kb[h] = kf[h].astype(jnp.bfloat16)
            vb[h] = vf[h].astype(jnp.bfloat16)
        dmas[2].wait()
        wob[:, :] = wof[:, :].astype(jnp.bfloat16)

        GLOB = 128


        def attn_glob_rows(nrows):
            for h in range(HQ_PER):
                qh = qbuf[0:nrows, h * DH:(h + 1) * DH]
                s = lax.dot_general(
                    qh, kb[h], (((1,), (1,)), ((), ())),
                    preferred_element_type=jnp.float32,
                ) * SCALE
                w = jnp.exp(s)
                w = w * (1.0 / jnp.sum(w, axis=1, keepdims=True))
                ctx_h = jnp.dot(w.astype(jnp.bfloat16), vb[h],
                                preferred_element_type=jnp.float32)
                cbuf[0:nrows, h * DH:(h + 1) * DH] = ctx_h.astype(jnp.bfloat16)

        def attn_win(off, r0, nrows, width, bs, with_glob):
            qi = lax.broadcasted_iota(jnp.int32, (nrows, width), 0) + off + r0
            kib = lax.broadcasted_iota(jnp.int32, (nrows, width), 1) + bs
            mask_b = (jnp.abs(qi - kib) <= 128) | (kib < 32)
            if with_glob:
                kig = lax.broadcasted_iota(jnp.int32, (nrows, GLOB), 1)
                mask_g = (kig < 32) & (bs != 0)
            for h in range(HQ_PER):
                qh = qbuf[pl.ds(off + r0, nrows), h * DH:(h + 1) * DH]
                s_b = lax.dot_general(
                    qh, kb[h, pl.ds(bs, width), :], (((1,), (1,)), ((), ())),
                    preferred_element_type=jnp.float32,
                ) * SCALE
                wb = jnp.exp(jnp.where(mask_b, s_b, NEG_INF))
                denom = jnp.sum(wb, axis=1, keepdims=True)
                if with_glob:
                    s_g = lax.dot_general(
                        qh, kb[h, 0:GLOB, :], (((1,), (1,)), ((), ())),
                        preferred_element_type=jnp.float32,
                    ) * SCALE
                    wg = jnp.exp(jnp.where(mask_g, s_g, NEG_INF))
                    denom = denom + jnp.sum(wg, axis=1, keepdims=True)
                r = 1.0 / denom
                ctx_h = jnp.dot(
                    (wb * r).astype(jnp.bfloat16), vb[h, pl.ds(bs, width), :],
                    preferred_element_type=jnp.float32,
                )
                if with_glob:
                    ctx_h = ctx_h + jnp.dot(
                        (wg * r).astype(jnp.bfloat16), vb[h, 0:GLOB, :],
                        preferred_element_type=jnp.float32,
                    )
                cbuf[r0:r0 + nrows, h * DH:(h + 1) * DH] = (
                    ctx_h.astype(jnp.bfloat16)
                )

        def compute_chunk(off):

            @pl.when(off == 0)
            def _chunk0():
                attn_glob_rows(32)
                attn_win(0, 32, CHUNK - 32, 512, 0, with_glob=False)

            @pl.when(off != 0)
            def _sparse():
                bs = jnp.minimum(off - 128, SKV - 512)
                attn_win(off, 0, CHUNK, 512, bs, with_glob=True)

            return jnp.dot(cbuf[:, :], wob[:, :],
                           preferred_element_type=jnp.float32)

        sendsA = []
        for j in range(N_DEV - 1):
            target = lax.rem(my_pos + 1 + j, N_DEV)
            sbufA[j] = compute_chunk(target * CHUNK).astype(jnp.bfloat16)
            rdma = pltpu.make_async_remote_copy(
                src_ref=sbufA.at[j],
                dst_ref=rbufA.at[2 - j],
                send_sem=sendA.at[j],
                recv_sem=recvA.at[2 - j],
                device_id=(target,),
                device_id_type=pl.DeviceIdType.MESH,
            )
            rdma.start()
            sendsA.append(rdma)

        pbuf[:, :] = compute_chunk(my_pos * CHUNK)

        for r in (2, 1, 0):
            recv = pltpu.make_async_remote_copy(
                src_ref=sbufA.at[0],
                dst_ref=rbufA.at[r],
                send_sem=sendA.at[0],
                recv_sem=recvA.at[r],
                device_id=(my_pos,),
                device_id_type=pl.DeviceIdType.MESH,
            )
            recv.wait_recv()
            pbuf[:, :] += rbufA[r].astype(jnp.float32)

        sbufB[:, :] = pbuf[:, :].astype(jnp.bfloat16)
        out_ref[0, pl.ds(my_pos * CHUNK, CHUNK), :] = sbufB[:, :]
        sendsB = []
        for j in (1, 0, 2):
            target = lax.rem(my_pos + 1 + j, N_DEV)
            rdma = pltpu.make_async_remote_copy(
                src_ref=sbufB,
                dst_ref=rbufB.at[2 - j],
                send_sem=sendB.at[j],
                recv_sem=recvB.at[2 - j],
                device_id=(target,),
                device_id_type=pl.DeviceIdType.MESH,
            )
            rdma.start()
            sendsB.append(rdma)

        for r in (2, 1, 0):
            recv = pltpu.make_async_remote_copy(
                src_ref=sbufB,
                dst_ref=rbufB.at[r],
                send_sem=sendB.at[0],
                recv_sem=recvB.at[r],
                device_id=(my_pos,),
                device_id_type=pl.DeviceIdType.MESH,
            )
            recv.wait_recv()
            src_dev = lax.rem(my_pos + 1 + r, N_DEV)
            out_ref[0, pl.ds(src_dev * CHUNK, CHUNK), :] = rbufB[r]

        for rdma in sendsA + sendsB:
            rdma.wait_send()

    return pl.pallas_call(
        body,
        out_shape=jax.ShapeDtypeStruct((1, SQ, D_MODEL), jnp.bfloat16),
        in_specs=[pl.BlockSpec(memory_space=pl.ANY)] * 5,
        out_specs=pl.BlockSpec(memory_space=pltpu.VMEM),
        scratch_shapes=[
            pltpu.VMEM((SQ, D_MODEL), jnp.float32),
            pltpu.VMEM((D_MODEL, HD_PER), jnp.float32),
            pltpu.VMEM((HD_PER, D_MODEL), jnp.float32),
            pltpu.VMEM((HQ_PER, SKV, DH), jnp.float32),
            pltpu.VMEM((HQ_PER, SKV, DH), jnp.float32),
            pltpu.VMEM((SQ, D_MODEL), jnp.bfloat16),
            pltpu.VMEM((D_MODEL, HD_PER), jnp.bfloat16),
            pltpu.VMEM((HD_PER, D_MODEL), jnp.bfloat16),
            pltpu.VMEM((HQ_PER, SKV, DH), jnp.bfloat16),
            pltpu.VMEM((HQ_PER, SKV, DH), jnp.bfloat16),
            pltpu.VMEM((SQ, HD_PER), jnp.bfloat16),
            pltpu.VMEM((CHUNK, HD_PER), jnp.bfloat16),
            pltpu.VMEM((CHUNK, D_MODEL), jnp.float32),
            pltpu.VMEM((N_DEV - 1, CHUNK, D_MODEL), jnp.bfloat16),
            pltpu.VMEM((N_DEV - 1, CHUNK, D_MODEL), jnp.bfloat16),
            pltpu.VMEM((CHUNK, D_MODEL), jnp.bfloat16),
            pltpu.VMEM((N_DEV - 1, CHUNK, D_MODEL), jnp.bfloat16),
            pltpu.SemaphoreType.DMA((5,)),
            pltpu.SemaphoreType.DMA((N_DEV - 1,)),
            pltpu.SemaphoreType.DMA((N_DEV - 1,)),
            pltpu.SemaphoreType.DMA((N_DEV - 1,)),
            pltpu.SemaphoreType.DMA((N_DEV - 1,)),
        ],
        compiler_params=pltpu.CompilerParams(
            collective_id=0, vmem_limit_bytes=64 * 1024 * 1024
        ),
    )(x, Wq, K_ext, V_ext, Wo)


# device time: 47687 ns/iter; 1.0846x vs baseline; 1.0752x over previous
import jax
import jax.numpy as jnp
from jax import lax
from jax.experimental import pallas as pl
from jax.experimental.pallas import tpu as pltpu

N_DEV = 4
SQ = 1024
SKV = 1024
D_MODEL = 1024
HQ_PER = 8
DH = 128
HD_PER = HQ_PER * DH
CHUNK = SQ // N_DEV
SCALE = 0.08838834764831843
NEG_INF = -1e9


def kernel(x, Wq, K_ext, V_ext, Wo):
    def body(x_hbm, wq_hbm, k_hbm, v_hbm, wo_hbm, out_ref,
             xf, wqf, wof, kf, vf, xb, wqb, wob, kb, vb,
             qbuf, cbuf, pbuf, sbufA, rbufA, sbufB, rbufB,
             copy_sems, sendA, recvA, sendB, recvB):
        my_pos = lax.axis_index("i")

        dmas = []
        for src, dst, i in (
            (x_hbm.at[0], xf, 0),
            (wq_hbm.at[:, pl.ds(my_pos * HD_PER, HD_PER)], wqf, 1),
            (wo_hbm.at[pl.ds(my_pos * HD_PER, HD_PER), :], wof, 2),
        ):
            c = pltpu.make_async_copy(src, dst, copy_sems.at[i])
            c.start()
            dmas.append(c)
        kv_dmas = []
        for h in range(HQ_PER):
            ck = pltpu.make_async_copy(
                k_hbm.at[0, :, h, :], kf.at[h], copy_sems.at[3]
            )
            cv = pltpu.make_async_copy(
                v_hbm.at[0, :, h, :], vf.at[h], copy_sems.at[4]
            )
            ck.start()
            cv.start()
            kv_dmas.append((ck, cv))

        barrier_sem = pltpu.get_barrier_semaphore()
        for j in range(N_DEV - 1):
            peer = lax.rem(my_pos + 1 + j, N_DEV)
            pl.semaphore_signal(barrier_sem, inc=1, device_id=(peer,),
                                device_id_type=pl.DeviceIdType.MESH)
        pl.semaphore_wait(barrier_sem, N_DEV - 1)

        dmas[0].wait()
        dmas[1].wait()
        xb[:, :] = xf[:, :].astype(jnp.bfloat16)
        wqb[:, :] = wqf[:, :].astype(jnp.bfloat16)
        qbuf[:, :] = (jnp.dot(
            xb[:, :], wqb[:, :], preferred_element_type=jnp.float32
        ) * SCALE).astype(jnp.bfloat16)

        for ck, cv in kv_dmas:
            ck.wait()
            cv.wait()
        for h in range(HQ_PER):
            kb[h] = kf[h].astype(jnp.bfloat16)
            vb[h] = vf[h].astype(jnp.bfloat16)
        dmas[2].wait()
        wob[:, :] = wof[:, :].astype(jnp.bfloat16)

        GLOB = 128


        def attn_glob_rows(nrows):
            for h in range(HQ_PER):
                qh = qbuf[0:nrows, h * DH:(h + 1) * DH]
                s = lax.dot_general(
                    qh, kb[h], (((1,), (1,)), ((), ())),
                    preferred_element_type=jnp.float32,
                )
                w = jnp.exp(s)
                r = 1.0 / jnp.sum(w, axis=1, keepdims=True)
                ctx_h = jnp.dot(w.astype(jnp.bfloat16), vb[h],
                                preferred_element_type=jnp.float32) * r
                cbuf[0:nrows, h * DH:(h + 1) * DH] = ctx_h.astype(jnp.bfloat16)

        def attn_win(off, r0, nrows, width, bs, with_glob):
            qi = lax.broadcasted_iota(jnp.int32, (nrows, width), 0) + off + r0
            kib = lax.broadcasted_iota(jnp.int32, (nrows, width), 1) + bs
            mask_b = (jnp.abs(qi - kib) <= 128) | (kib < 32)
            if with_glob:
                kig = lax.broadcasted_iota(jnp.int32, (nrows, GLOB), 1)
                mask_g = (kig < 32) & (bs != 0)
            for h in range(HQ_PER):
                qh = qbuf[pl.ds(off + r0, nrows), h * DH:(h + 1) * DH]
                s_b = lax.dot_general(
                    qh, kb[h, pl.ds(bs, width), :], (((1,), (1,)), ((), ())),
                    preferred_element_type=jnp.float32,
                )
                wb = jnp.exp(jnp.where(mask_b, s_b, NEG_INF))
                denom = jnp.sum(wb, axis=1, keepdims=True)
                if with_glob:
                    s_g = lax.dot_general(
                        qh, kb[h, 0:GLOB, :], (((1,), (1,)), ((), ())),
                        preferred_element_type=jnp.float32,
                    )
                    wg = jnp.exp(jnp.where(mask_g, s_g, NEG_INF))
                    denom = denom + jnp.sum(wg, axis=1, keepdims=True)
                r = 1.0 / denom
                ctx_h = jnp.dot(
                    wb.astype(jnp.bfloat16), vb[h, pl.ds(bs, width), :],
                    preferred_element_type=jnp.float32,
                )
                if with_glob:
                    ctx_h = ctx_h + jnp.dot(
                        wg.astype(jnp.bfloat16), vb[h, 0:GLOB, :],
                        preferred_element_type=jnp.float32,
                    )
                ctx_h = ctx_h * r
                cbuf[r0:r0 + nrows, h * DH:(h + 1) * DH] = (
                    ctx_h.astype(jnp.bfloat16)
                )

        def compute_chunk(off):

            @pl.when(off == 0)
            def _chunk0():
                attn_glob_rows(32)
                attn_win(0, 32, CHUNK - 32, 512, 0, with_glob=False)

            @pl.when(off != 0)
            def _sparse():
                bs = jnp.minimum(off - 128, SKV - 512)
                attn_win(off, 0, CHUNK, 512, bs, with_glob=True)

            return jnp.dot(cbuf[:, :], wob[:, :],
                           preferred_element_type=jnp.float32)

        sendsA = []
        for j in range(N_DEV - 1):
            target = lax.rem(my_pos + 1 + j, N_DEV)
            sbufA[j] = compute_chunk(target * CHUNK).astype(jnp.bfloat16)
            rdma = pltpu.make_async_remote_copy(
                src_ref=sbufA.at[j],
                dst_ref=rbufA.at[2 - j],
                send_sem=sendA.at[j],
                recv_sem=recvA.at[2 - j],
                device_id=(target,),
                device_id_type=pl.DeviceIdType.MESH,
            )
            rdma.start()
            sendsA.append(rdma)

        pbuf[:, :] = compute_chunk(my_pos * CHUNK)

        for r in (2, 1, 0):
            recv = pltpu.make_async_remote_copy(
                src_ref=sbufA.at[0],
                dst_ref=rbufA.at[r],
                send_sem=sendA.at[0],
                recv_sem=recvA.at[r],
                device_id=(my_pos,),
                device_id_type=pl.DeviceIdType.MESH,
            )
            recv.wait_recv()
            pbuf[:, :] += rbufA[r].astype(jnp.float32)

        sbufB[:, :] = pbuf[:, :].astype(jnp.bfloat16)
        out_ref[0, pl.ds(my_pos * CHUNK, CHUNK), :] = sbufB[:, :]
        sendsB = []
        for j in (1, 0, 2):
            target = lax.rem(my_pos + 1 + j, N_DEV)
            rdma = pltpu.make_async_remote_copy(
                src_ref=sbufB,
                dst_ref=rbufB.at[2 - j],
                send_sem=sendB.at[j],
                recv_sem=recvB.at[2 - j],
                device_id=(target,),
                device_id_type=pl.DeviceIdType.MESH,
            )
            rdma.start()
            sendsB.append(rdma)

        for r in (2, 1, 0):
            recv = pltpu.make_async_remote_copy(
                src_ref=sbufB,
                dst_ref=rbufB.at[r],
                send_sem=sendB.at[0],
                recv_sem=recvB.at[r],
                device_id=(my_pos,),
                device_id_type=pl.DeviceIdType.MESH,
            )
            recv.wait_recv()
            src_dev = lax.rem(my_pos + 1 + r, N_DEV)
            out_ref[0, pl.ds(src_dev * CHUNK, CHUNK), :] = rbufB[r]

        for rdma in sendsA + sendsB:
            rdma.wait_send()

    return pl.pallas_call(
        body,
        out_shape=jax.ShapeDtypeStruct((1, SQ, D_MODEL), jnp.bfloat16),
        in_specs=[pl.BlockSpec(memory_space=pl.ANY)] * 5,
        out_specs=pl.BlockSpec(memory_space=pltpu.VMEM),
        scratch_shapes=[
            pltpu.VMEM((SQ, D_MODEL), jnp.float32),
            pltpu.VMEM((D_MODEL, HD_PER), jnp.float32),
            pltpu.VMEM((HD_PER, D_MODEL), jnp.float32),
            pltpu.VMEM((HQ_PER, SKV, DH), jnp.float32),
            pltpu.VMEM((HQ_PER, SKV, DH), jnp.float32),
            pltpu.VMEM((SQ, D_MODEL), jnp.bfloat16),
            pltpu.VMEM((D_MODEL, HD_PER), jnp.bfloat16),
            pltpu.VMEM((HD_PER, D_MODEL), jnp.bfloat16),
            pltpu.VMEM((HQ_PER, SKV, DH), jnp.bfloat16),
            pltpu.VMEM((HQ_PER, SKV, DH), jnp.bfloat16),
            pltpu.VMEM((SQ, HD_PER), jnp.bfloat16),
            pltpu.VMEM((CHUNK, HD_PER), jnp.bfloat16),
            pltpu.VMEM((CHUNK, D_MODEL), jnp.float32),
            pltpu.VMEM((N_DEV - 1, CHUNK, D_MODEL), jnp.bfloat16),
            pltpu.VMEM((N_DEV - 1, CHUNK, D_MODEL), jnp.bfloat16),
            pltpu.VMEM((CHUNK, D_MODEL), jnp.bfloat16),
            pltpu.VMEM((N_DEV - 1, CHUNK, D_MODEL), jnp.bfloat16),
            pltpu.SemaphoreType.DMA((5,)),
            pltpu.SemaphoreType.DMA((N_DEV - 1,)),
            pltpu.SemaphoreType.DMA((N_DEV - 1,)),
            pltpu.SemaphoreType.DMA((N_DEV - 1,)),
            pltpu.SemaphoreType.DMA((N_DEV - 1,)),
        ],
        compiler_params=pltpu.CompilerParams(
            collective_id=0, vmem_limit_bytes=64 * 1024 * 1024
        ),
    )(x, Wq, K_ext, V_ext, Wo)
